# Initial kernel scaffold; baseline (speedup 1.0000x reference)
import jax
import jax.numpy as jnp
from jax import lax
from jax.experimental import pallas as pl
from jax.experimental.pallas import tpu as pltpu
from jax.experimental.pallas import tpu_sc as plsc

NC = 2
NS = 16
NW = NC * NS

N = 10000
E = 320000
H = 128
G = 64
C = 10

EPT = E // NW
K = 80
NCHUNK = EPT // K
RPT = N // NS
ZROWS = 125

_f32 = jnp.float32
_i32 = jnp.int32


def _worker_id():
    return lax.axis_index("s") * NC + lax.axis_index("c")


def _deg_body(col3, w3, deg_out, col_buf, w_buf, zbuf, deg_acc):
    cid = lax.axis_index("c")
    sid = lax.axis_index("s")
    wid = _worker_id()

    @pl.when(sid == 0)
    def _zero():
        def z(i, carry):
            zbuf[pl.ds(i * 16, 16)] = jnp.zeros((16,), _f32)
            return carry
        lax.fori_loop(0, 125, z, 0)
        for t in range(5):
            pltpu.sync_copy(zbuf, deg_acc.at[pl.ds(t * 2000, 2000)])

    plsc.subcore_barrier()

    pltpu.sync_copy(col3.at[wid], col_buf)
    pltpu.sync_copy(w3.at[wid], w_buf)

    def body(j, carry):
        pltpu.sync_copy(w_buf.at[j], deg_acc.at[col_buf.at[j]], add=True)
        return carry
    lax.fori_loop(0, NCHUNK, body, 0)

    plsc.subcore_barrier()

    @pl.when(sid == 0)
    def _write():
        pltpu.sync_copy(deg_acc, deg_out.at[cid])


def _deg_call(col3, w3):
    return pl.kernel(
        _deg_body,
        out_type=jax.ShapeDtypeStruct((NC, N), _f32),
        mesh=plsc.VectorSubcoreMesh(core_axis_name="c", subcore_axis_name="s"),
        scratch_types=[
            pltpu.VMEM((NCHUNK, K), _i32),
            pltpu.VMEM((NCHUNK, K), _f32),
            pltpu.VMEM((2000,), _f32),
            pltpu.VMEM_SHARED((N,), _f32),
        ],
    )(col3, w3)


def _norm_body(row_h, col_h, w_h, dinv_h, norm_out,
               dinv_buf, row_buf, col_buf, w_buf, norm_buf):
    wid = _worker_id()
    base = wid * EPT
    pltpu.sync_copy(dinv_h, dinv_buf)
    pltpu.sync_copy(row_h.at[pl.ds(base, EPT)], row_buf)
    pltpu.sync_copy(col_h.at[pl.ds(base, EPT)], col_buf)
    pltpu.sync_copy(w_h.at[pl.ds(base, EPT)], w_buf)

    def body(i, carry):
        sl = pl.ds(i * 16, 16)
        vr = plsc.load_gather(dinv_buf, [row_buf[sl]])
        vc = plsc.load_gather(dinv_buf, [col_buf[sl]])
        norm_buf[sl] = vr * w_buf[sl] * vc
        return carry
    lax.fori_loop(0, EPT // 16, body, 0)

    pltpu.sync_copy(norm_buf, norm_out.at[pl.ds(base, EPT)])


def _norm_call(row, col, w, dinv):
    return pl.kernel(
        _norm_body,
        out_type=jax.ShapeDtypeStruct((E,), _f32),
        mesh=plsc.VectorSubcoreMesh(core_axis_name="c", subcore_axis_name="s"),
        scratch_types=[
            pltpu.VMEM((N,), _f32),
            pltpu.VMEM((EPT,), _i32),
            pltpu.VMEM((EPT,), _i32),
            pltpu.VMEM((EPT,), _f32),
            pltpu.VMEM((EPT,), _f32),
        ],
    )(row, col, w, dinv)


def _agg_body(xw_h, row3, col3, norm_h, agg_out,
              row_buf, col_buf, norm_buf, gbuf, zbuf, acc, sem):
    cid = lax.axis_index("c")
    sid = lax.axis_index("s")
    wid = _worker_id()

    def z(i, carry):
        for q in range(8):
            zbuf[i, pl.ds(q * 16, 16)] = jnp.zeros((16,), _f32)
        return carry
    lax.fori_loop(0, ZROWS, z, 0)
    for t in range(RPT // ZROWS):
        pltpu.sync_copy(zbuf, acc.at[pl.ds(sid * RPT + t * ZROWS, ZROWS)])

    plsc.subcore_barrier()

    pltpu.sync_copy(row3.at[wid], row_buf)
    pltpu.sync_copy(col3.at[wid], col_buf)
    pltpu.sync_copy(norm_h.at[pl.ds(wid * EPT, EPT)], norm_buf)

    def chunk(j, carry):
        pltpu.async_copy(xw_h.at[row_buf.at[j]], gbuf, sem).wait()

        def scale(e, c2):
            splat = plsc.load_gather(norm_buf, [jnp.full((16,), j * K + e, _i32)])
            for q in range(8):
                sl = pl.ds(q * 16, 16)
                gbuf[e, sl] = gbuf[e, sl] * splat
            return c2
        lax.fori_loop(0, K, scale, 0)

        pltpu.sync_copy(gbuf, acc.at[col_buf.at[j]], add=True)
        return carry
    lax.fori_loop(0, NCHUNK, chunk, 0)

    plsc.subcore_barrier()

    pltpu.sync_copy(acc.at[pl.ds(sid * RPT, RPT)],
                    agg_out.at[cid, pl.ds(sid * RPT, RPT)])


def _agg_call(xw, row3, col3, norm):
    return pl.kernel(
        _agg_body,
        out_type=jax.ShapeDtypeStruct((NC, N, H), _f32),
        mesh=plsc.VectorSubcoreMesh(core_axis_name="c", subcore_axis_name="s"),
        scratch_types=[
            pltpu.VMEM((NCHUNK, K), _i32),
            pltpu.VMEM((NCHUNK, K), _i32),
            pltpu.VMEM((EPT,), _f32),
            pltpu.VMEM((K, H), _f32),
            pltpu.VMEM((ZROWS, H), _f32),
            pltpu.VMEM_SHARED((N, H), _f32),
            pltpu.SemaphoreType.DMA,
        ],
    )(xw, row3, col3, norm)


def _prep_body(deg2_ref, x_ref, w0_ref, dinv_ref, dinvc_ref, xw_ref):
    d = deg2_ref[0, :] + deg2_ref[1, :] + 1.0
    dinv = lax.rsqrt(jnp.maximum(d, 1e-12))
    dinv_ref[...] = dinv
    dinvc_ref[...] = dinv.reshape(N, 1)
    xw_ref[...] = jnp.dot(x_ref[...], w0_ref[...], preferred_element_type=_f32)


def _prep_call(deg2, x, W0):
    return pl.pallas_call(
        _prep_body,
        out_shape=[
            jax.ShapeDtypeStruct((N,), _f32),
            jax.ShapeDtypeStruct((N, 1), _f32),
            jax.ShapeDtypeStruct((N, H), _f32),
        ],
    )(deg2, x, W0)


def _bn_relu(agg_ref, xw_ref, dinvc_ref, b_ref, g_ref, be_ref):
    dc = dinvc_ref[...]
    z = agg_ref[0] + agg_ref[1] + dc * dc * xw_ref[...] + b_ref[...]
    mean = jnp.mean(z, axis=0, keepdims=True)
    zc = z - mean
    var = jnp.mean(zc * zc, axis=0, keepdims=True)
    h = g_ref[...] * zc * lax.rsqrt(var + 1e-5) + be_ref[...]
    return jnp.maximum(h, 0.0)


def _post_body(agg_ref, xw_ref, dinvc_ref, b_ref, g_ref, be_ref, wn_ref, out_ref):
    h = _bn_relu(agg_ref, xw_ref, dinvc_ref, b_ref, g_ref, be_ref)
    out_ref[...] = jnp.dot(h, wn_ref[...], preferred_element_type=_f32)


def _post_call(agg, xw, dinvc, b, gamma, beta, Wn):
    return pl.pallas_call(
        _post_body,
        out_shape=jax.ShapeDtypeStruct((N, H), _f32),
    )(agg, xw, dinvc, b, gamma, beta, Wn)


def _final_body(agg_ref, xw_ref, dinvc_ref, b_ref, g_ref, be_ref, batch_ref,
                mw0_ref, mb0_ref, mw1_ref, mb1_ref, mw2_ref, mb2_ref, out_ref):
    h = _bn_relu(agg_ref, xw_ref, dinvc_ref, b_ref, g_ref, be_ref)
    oh = (lax.broadcasted_iota(_i32, (G, N), 0) == batch_ref[...][None, :])
    oh = oh.astype(_f32)
    summ = jnp.dot(oh, h, preferred_element_type=_f32)
    cnt = jnp.dot(oh, jnp.ones((N, 1), _f32), preferred_element_type=_f32)
    pooled = summ / jnp.maximum(cnt, 1.0)
    y = jnp.dot(pooled, mw0_ref[...], preferred_element_type=_f32) + mb0_ref[...]
    y = jnp.maximum(y, 0.0)
    y = jnp.dot(y, mw1_ref[...], preferred_element_type=_f32) + mb1_ref[...]
    y = jnp.maximum(y, 0.0)
    y = jnp.dot(y, mw2_ref[...], preferred_element_type=_f32) + mb2_ref[...]
    m = jnp.max(y, axis=0, keepdims=True)
    lse = jnp.log(jnp.sum(jnp.exp(y - m), axis=0, keepdims=True)) + m
    out_ref[...] = y - lse


def _final_call(agg, xw, dinvc, b, gamma, beta, batch, mW0, mb0, mW1, mb1, mW2, mb2):
    return pl.pallas_call(
        _final_body,
        out_shape=jax.ShapeDtypeStruct((G, C), _f32),
    )(agg, xw, dinvc, b, gamma, beta, batch, mW0, mb0, mW1, mb1, mW2, mb2)


def kernel(x, edge_index, edge_attr, batch, W0, b0, W1, b1, W2, b2, W3, b3,
           gamma, beta, mW0, mb0, mW1, mb1, mW2, mb2):
    row = edge_index[0]
    col = edge_index[1]
    row3 = row.reshape(NW, NCHUNK, K)
    col3 = col.reshape(NW, NCHUNK, K)
    w3 = edge_attr.reshape(NW, NCHUNK, K)

    deg2 = _deg_call(col3, w3)
    dinv, dinvc, xw = _prep_call(deg2, x, W0)
    norm = _norm_call(row, col, edge_attr, dinv)

    Ws = [W1, W2, W3]
    bs = [b0, b1, b2, b3]
    for layer in range(4):
        agg = _agg_call(xw, row3, col3, norm)
        if layer < 3:
            xw = _post_call(agg, xw, dinvc, bs[layer], gamma, beta, Ws[layer])
        else:
            out = _final_call(agg, xw, dinvc, bs[layer], gamma, beta, batch,
                              mW0, mb0, mW1, mb1, mW2, mb2)
    return out


# speedup vs baseline: 6.8167x; 6.8167x over previous
import jax
import jax.numpy as jnp
from jax import lax
from jax.experimental import pallas as pl
from jax.experimental.pallas import tpu as pltpu
from jax.experimental.pallas import tpu_sc as plsc

NC = 2
NS = 16
NW = NC * NS

N = 10000
E = 320000
H = 128
G = 64
C = 10

K = 128
NCHUNK = 80
EPT = K * NCHUNK
EPAD = EPT * NW

RB = 624
ZR = 48

K2 = 80
NCHUNK2 = EPT // K2
NP = 10240
RBP = NP // NS

_f32 = jnp.float32
_i32 = jnp.int32

_MESH = dict(core_axis_name="c", subcore_axis_name="s", num_cores=NC,
             num_subcores=NS)
_PARAMS = pltpu.CompilerParams(needs_layout_passes=False)


def _worker_id():
    return lax.axis_index("s") * NC + lax.axis_index("c")


def _deg_body(col3, w3, deg_out, col_buf, w_buf, zbuf, deg_stage, deg_acc):
    cid = lax.axis_index("c")
    sid = lax.axis_index("s")
    wid = _worker_id()

    @pl.when(sid == 0)
    def _zero():
        def z(i, carry):
            zbuf[pl.ds(i * 16, 16)] = jnp.zeros((16,), _f32)
            return carry
        lax.fori_loop(0, 125, z, 0)
        for t in range(5):
            pltpu.sync_copy(zbuf, deg_acc.at[pl.ds(t * 2000, 2000)])

    plsc.subcore_barrier()

    pltpu.sync_copy(col3.at[wid], col_buf)
    pltpu.sync_copy(w3.at[wid], w_buf)

    def body(j, carry):
        pltpu.sync_copy(w_buf.at[j], deg_acc.at[col_buf.at[j]], add=True)
        return carry
    lax.fori_loop(0, NCHUNK, body, 0)

    plsc.subcore_barrier()

    dbase = sid * RB
    pltpu.sync_copy(deg_acc.at[pl.ds(dbase, RB)], deg_stage)
    pltpu.sync_copy(deg_stage, deg_out.at[pl.ds(cid * N + dbase, RB)])

    @pl.when(sid == NS - 1)
    def _write_tail():
        tail = N - NS * RB
        pltpu.sync_copy(deg_acc.at[pl.ds(NS * RB, tail)],
                        deg_stage.at[pl.ds(0, tail)])
        pltpu.sync_copy(deg_stage.at[pl.ds(0, tail)],
                        deg_out.at[pl.ds(cid * N + NS * RB, tail)])


def _deg_call(col3, w3):
    return pl.kernel(
        _deg_body,
        out_type=jax.ShapeDtypeStruct((NC * N,), _f32),
        mesh=plsc.VectorSubcoreMesh(**_MESH),
        compiler_params=_PARAMS,
        scratch_types=[
            pltpu.VMEM((NCHUNK, K), _i32),
            pltpu.VMEM((NCHUNK, K), _f32),
            pltpu.VMEM((2000,), _f32),
            pltpu.VMEM((RB,), _f32),
            pltpu.VMEM_SHARED((N,), _f32),
        ],
    )(col3, w3)


def _norm_body(row_h, col_h, w_h, dinv_h, norm_out,
               dinv_buf, row_buf, col_buf, w_buf, norm_buf):
    wid = _worker_id()
    base = wid * EPT
    pltpu.sync_copy(dinv_h, dinv_buf)
    pltpu.sync_copy(row_h.at[pl.ds(base, EPT)], row_buf)
    pltpu.sync_copy(col_h.at[pl.ds(base, EPT)], col_buf)
    pltpu.sync_copy(w_h.at[pl.ds(base, EPT)], w_buf)

    def body(i, carry):
        sl = pl.ds(i * 16, 16)
        vr = plsc.load_gather(dinv_buf, [row_buf[sl]])
        vc = plsc.load_gather(dinv_buf, [col_buf[sl]])
        norm_buf[sl] = vr * w_buf[sl] * vc
        return carry
    lax.fori_loop(0, EPT // 16, body, 0)

    pltpu.sync_copy(norm_buf, norm_out.at[pl.ds(base, EPT)])


def _norm_call(row, col, w, dinv):
    return pl.kernel(
        _norm_body,
        out_type=jax.ShapeDtypeStruct((EPAD,), _f32),
        mesh=plsc.VectorSubcoreMesh(**_MESH),
        compiler_params=_PARAMS,
        scratch_types=[
            pltpu.VMEM((N,), _f32),
            pltpu.VMEM((EPT,), _i32),
            pltpu.VMEM((EPT,), _i32),
            pltpu.VMEM((EPT,), _f32),
            pltpu.VMEM((EPT,), _f32),
        ],
    )(row, col, w, dinv)


def _agg_body(xw_h, row_h, col_h, norm_h, agg_out,
              row_buf, col_buf, nbuf0, nbuf1, gbuf0, gbuf1, acc,
              sem0, sem1, nsem0, nsem1):
    cid = lax.axis_index("c")
    sid = lax.axis_index("s")
    wid = _worker_id()
    base = wid * EPT
    rbase = sid * RBP

    def z(i, carry):
        for q in range(8):
            gbuf0[i, pl.ds(q * 16, 16)] = jnp.zeros((16,), _f32)
        return carry
    lax.fori_loop(0, K2, z, 0)
    for t in range(RBP // K2):
        pltpu.sync_copy(gbuf0, acc.at[pl.ds(rbase + t * K2, K2)])

    plsc.subcore_barrier()

    pltpu.sync_copy(row_h.at[pl.ds(base, EPT)], row_buf)
    pltpu.sync_copy(col_h.at[pl.ds(base, EPT)], col_buf)

    def scale_scatter(j, gbuf, nbuf, nsem):
        pltpu.make_async_copy(norm_h.at[pl.ds(base + j * K2, K2)],
                              nbuf, nsem).wait()

        def group(g, c2):
            nv = nbuf[pl.ds(g * 16, 16)]
            for e0 in range(16):
                splat = jnp.full((16,), nv[e0], _f32)
                for q in range(8):
                    sl = pl.ds(q * 16, 16)
                    gbuf[g * 16 + e0, sl] = gbuf[g * 16 + e0, sl] * splat
            return c2
        lax.fori_loop(0, K2 // 16, group, 0)
        pltpu.sync_copy(gbuf, acc.at[col_buf.at[pl.ds(j * K2, K2)]], add=True)

    pltpu.async_copy(xw_h.at[row_buf.at[pl.ds(0, K2)]], gbuf0, sem0)
    pltpu.async_copy(norm_h.at[pl.ds(base, K2)], nbuf0, nsem0)

    def pair(i, carry):
        j0 = 2 * i
        pltpu.make_async_copy(xw_h.at[row_buf.at[pl.ds(j0 * K2, K2)]],
                              gbuf0, sem0).wait()
        pltpu.async_copy(xw_h.at[row_buf.at[pl.ds((j0 + 1) * K2, K2)]],
                         gbuf1, sem1)
        pltpu.async_copy(norm_h.at[pl.ds(base + (j0 + 1) * K2, K2)],
                         nbuf1, nsem1)
        scale_scatter(j0, gbuf0, nbuf0, nsem0)
        pltpu.make_async_copy(xw_h.at[row_buf.at[pl.ds((j0 + 1) * K2, K2)]],
                              gbuf1, sem1).wait()

        @pl.when(i < NCHUNK2 // 2 - 1)
        def _issue_next():
            pltpu.async_copy(xw_h.at[row_buf.at[pl.ds((j0 + 2) * K2, K2)]],
                             gbuf0, sem0)
            pltpu.async_copy(norm_h.at[pl.ds(base + (j0 + 2) * K2, K2)],
                             nbuf0, nsem0)

        scale_scatter(j0 + 1, gbuf1, nbuf1, nsem1)
        return carry
    lax.fori_loop(0, NCHUNK2 // 2, pair, 0)

    plsc.subcore_barrier()

    nst = RBP // K2
    for t in range(nst):
        g = gbuf0 if t % 2 == 0 else gbuf1
        sem = sem0 if t % 2 == 0 else sem1
        if t >= 2:
            pltpu.make_async_copy(
                g, agg_out.at[cid, pl.ds(rbase + (t - 2) * K2, K2)], sem
            ).wait()
        pltpu.sync_copy(acc.at[pl.ds(rbase + t * K2, K2)], g)
        pltpu.async_copy(g, agg_out.at[cid, pl.ds(rbase + t * K2, K2)], sem)
    for t in range(nst - 2, nst):
        g = gbuf0 if t % 2 == 0 else gbuf1
        sem = sem0 if t % 2 == 0 else sem1
        pltpu.make_async_copy(
            g, agg_out.at[cid, pl.ds(rbase + t * K2, K2)], sem
        ).wait()


def _agg_call(xw, row, col, norm):
    return pl.kernel(
        _agg_body,
        out_type=jax.ShapeDtypeStruct((NC, NP, H), _f32),
        mesh=plsc.VectorSubcoreMesh(**_MESH),
        compiler_params=_PARAMS,
        scratch_types=[
            pltpu.VMEM((EPT,), _i32),
            pltpu.VMEM((EPT,), _i32),
            pltpu.VMEM((K2,), _f32),
            pltpu.VMEM((K2,), _f32),
            pltpu.VMEM((K2, H), _f32),
            pltpu.VMEM((K2, H), _f32),
            pltpu.VMEM_SHARED((NP, H), _f32),
            pltpu.SemaphoreType.DMA,
            pltpu.SemaphoreType.DMA,
            pltpu.SemaphoreType.DMA,
            pltpu.SemaphoreType.DMA,
        ],
    )(xw, row, col, norm)


def _prep_body(deg2_ref, x_ref, w0_ref, dinv_ref, dinvc_ref, xw_ref):
    d = deg2_ref[0, :] + deg2_ref[1, :] + 1.0
    dinv = lax.rsqrt(jnp.maximum(d, 1e-12))
    dinv_ref[...] = dinv
    dinvc_ref[...] = dinv.reshape(N, 1)
    xw_ref[...] = jnp.dot(x_ref[...], w0_ref[...], preferred_element_type=_f32)


def _prep_call(deg2, x, W0):
    return pl.pallas_call(
        _prep_body,
        out_shape=[
            jax.ShapeDtypeStruct((N,), _f32),
            jax.ShapeDtypeStruct((N, 1), _f32),
            jax.ShapeDtypeStruct((N, H), _f32),
        ],
    )(deg2, x, W0)


def _bn_relu(agg_ref, xw_ref, dinvc_ref, b_ref, g_ref, be_ref):
    dc = dinvc_ref[...]
    z = agg_ref[0, :N, :] + agg_ref[1, :N, :] + dc * dc * xw_ref[...] + b_ref[...]
    mean = jnp.mean(z, axis=0, keepdims=True)
    zc = z - mean
    var = jnp.mean(zc * zc, axis=0, keepdims=True)
    h = g_ref[...] * zc * lax.rsqrt(var + 1e-5) + be_ref[...]
    return jnp.maximum(h, 0.0)


def _post_body(agg_ref, xw_ref, dinvc_ref, b_ref, g_ref, be_ref, wn_ref, out_ref):
    h = _bn_relu(agg_ref, xw_ref, dinvc_ref, b_ref, g_ref, be_ref)
    out_ref[...] = jnp.dot(h, wn_ref[...], preferred_element_type=_f32)


def _post_call(agg, xw, dinvc, b, gamma, beta, Wn):
    return pl.pallas_call(
        _post_body,
        out_shape=jax.ShapeDtypeStruct((N, H), _f32),
    )(agg, xw, dinvc, b, gamma, beta, Wn)


def _final_body(agg_ref, xw_ref, dinvc_ref, b_ref, g_ref, be_ref, batch_ref,
                mw0_ref, mb0_ref, mw1_ref, mb1_ref, mw2_ref, mb2_ref, out_ref):
    h = _bn_relu(agg_ref, xw_ref, dinvc_ref, b_ref, g_ref, be_ref)
    oh = (lax.broadcasted_iota(_i32, (G, N), 0) == batch_ref[...][None, :])
    oh = oh.astype(_f32)
    summ = jnp.dot(oh, h, preferred_element_type=_f32)
    cnt = jnp.dot(oh, jnp.ones((N, 1), _f32), preferred_element_type=_f32)
    pooled = summ / jnp.maximum(cnt, 1.0)
    y = jnp.dot(pooled, mw0_ref[...], preferred_element_type=_f32) + mb0_ref[...]
    y = jnp.maximum(y, 0.0)
    y = jnp.dot(y, mw1_ref[...], preferred_element_type=_f32) + mb1_ref[...]
    y = jnp.maximum(y, 0.0)
    y = jnp.dot(y, mw2_ref[...], preferred_element_type=_f32) + mb2_ref[...]
    m = jnp.max(y, axis=0, keepdims=True)
    lse = jnp.log(jnp.sum(jnp.exp(y - m), axis=0, keepdims=True)) + m
    out_ref[...] = y - lse


def _final_call(agg, xw, dinvc, b, gamma, beta, batch, mW0, mb0, mW1, mb1, mW2, mb2):
    return pl.pallas_call(
        _final_body,
        out_shape=jax.ShapeDtypeStruct((G, C), _f32),
    )(agg, xw, dinvc, b, gamma, beta, batch, mW0, mb0, mW1, mb1, mW2, mb2)


def kernel(x, edge_index, edge_attr, batch, W0, b0, W1, b1, W2, b2, W3, b3,
           gamma, beta, mW0, mb0, mW1, mb1, mW2, mb2):
    pad = EPAD - E
    row = jnp.concatenate([edge_index[0], jnp.zeros((pad,), _i32)])
    col = jnp.concatenate([edge_index[1], jnp.zeros((pad,), _i32)])
    w = jnp.concatenate([edge_attr, jnp.zeros((pad,), _f32)])
    row3 = row.reshape(NW, NCHUNK, K)
    col3 = col.reshape(NW, NCHUNK, K)
    w3 = w.reshape(NW, NCHUNK, K)

    deg2 = _deg_call(col3, w3).reshape(NC, N)
    dinv, dinvc, xw = _prep_call(deg2, x, W0)
    norm = _norm_call(row, col, w, dinv)

    Ws = [W1, W2, W3]
    bs = [b0, b1, b2, b3]
    for layer in range(4):
        agg = _agg_call(xw, row, col, norm)
        if layer < 3:
            xw = _post_call(agg, xw, dinvc, bs[layer], gamma, beta, Ws[layer])
        else:
            out = _final_call(agg, xw, dinvc, bs[layer], gamma, beta, batch,
                              mW0, mb0, mW1, mb1, mW2, mb2)
    return out
